# Initial kernel scaffold; baseline (speedup 1.0000x reference)
import jax
import jax.numpy as jnp
from jax.experimental import pallas as pl


def kernel(context_trajectory, motif_keys, motif_values, epi_keys, epi_values):
    raise NotImplementedError("write your pallas kernel here")


# speedup vs baseline: 1.7684x; 1.7684x over previous
import functools

import jax
import jax.numpy as jnp
from jax.experimental import pallas as pl


def _body(scale, ctx_ref, mk_ref, mv_ref, ek_ref, ev_ref,
          comb_ref, eattn_ref, mattn_ref):
    ctx = ctx_ref[...]
    ms = jax.lax.dot_general(
        ctx, mk_ref[...], (((1,), (1,)), ((), ())),
        preferred_element_type=jnp.float32) * scale
    ms = ms - jnp.max(ms, axis=-1, keepdims=True)
    me = jnp.exp(ms)
    m_attn = me / jnp.sum(me, axis=-1, keepdims=True)
    m_read = jax.lax.dot_general(
        m_attn, mv_ref[...], (((1,), (0,)), ((), ())),
        preferred_element_type=jnp.float32)
    es = jax.lax.dot_general(
        m_read, ek_ref[...], (((1,), (1,)), ((), ())),
        preferred_element_type=jnp.float32) * scale
    es = es - jnp.max(es, axis=-1, keepdims=True)
    ee = jnp.exp(es)
    e_attn = ee / jnp.sum(ee, axis=-1, keepdims=True)
    e_read = jax.lax.dot_general(
        e_attn, ev_ref[...], (((1,), (0,)), ((), ())),
        preferred_element_type=jnp.float32)

    d = ctx.shape[1]
    comb_ref[:, :d] = e_read
    comb_ref[:, d:] = m_read
    eattn_ref[...] = e_attn
    mattn_ref[...] = m_attn


def kernel(context_trajectory, motif_keys, motif_values, epi_keys, epi_values):
    B, d = context_trajectory.shape
    M = motif_keys.shape[0]
    N = epi_keys.shape[0]
    scale = 1.0 / (float(d) ** 0.5)
    bB = 128
    grid = (B // bB,)

    full = lambda i: (0, 0)
    row = lambda i: (i, 0)

    out = pl.pallas_call(
        functools.partial(_body, scale),
        grid=grid,
        in_specs=[
            pl.BlockSpec((bB, d), row),
            pl.BlockSpec((M, d), full),
            pl.BlockSpec((M, d), full),
            pl.BlockSpec((N, d), full),
            pl.BlockSpec((N, d), full),
        ],
        out_specs=[
            pl.BlockSpec((bB, 2 * d), row),
            pl.BlockSpec((bB, N), row),
            pl.BlockSpec((bB, M), row),
        ],
        out_shape=[
            jax.ShapeDtypeStruct((B, 2 * d), jnp.float32),
            jax.ShapeDtypeStruct((B, N), jnp.float32),
            jax.ShapeDtypeStruct((B, M), jnp.float32),
        ],
    )(context_trajectory, motif_keys, motif_values, epi_keys, epi_values)
    return tuple(out)
